# Initial kernel scaffold; baseline (speedup 1.0000x reference)
import jax
import jax.numpy as jnp
from jax.experimental import pallas as pl


def kernel(x, edge_index, W1, a_src1, a_dst1, W2, a_src2, a_dst2):
    raise NotImplementedError("write your pallas kernel here")


# speedup vs baseline: 12.9873x; 12.9873x over previous
import functools

import jax
import jax.numpy as jnp
from jax import lax
from jax.experimental import pallas as pl
from jax.experimental.pallas import tpu as pltpu
from jax.experimental.pallas import tpu_sc as plsc

N = 10000
E_RAW = 320000
E_TOT = E_RAW + N
D1 = 128
D2 = 64

NC = 2
NS = 16
L = 16
NW = NC * NS
E_PAD = ((E_TOT + NW * L - 1) // (NW * L)) * (NW * L)
PER_TILE = E_PAD // NW
GROUPS = PER_TILE // L
ROWS_PER_TILE = N // NS
ZCH = 25
NCHUNK = 5
CH = PER_TILE // NCHUNK
CGROUPS = CH // L


def _ones_cols(n):
    cols = lax.broadcasted_iota(jnp.int32, (n, L), 1)
    return jnp.where(cols == 0, 1.0, 0.0).astype(jnp.float32)


def _prep1_body(x_ref, w_ref, a2_ref, hlo_ref, hhi_ref, alph_ref):
    x = x_ref[...]
    h = jnp.dot(x, w_ref[...], preferred_element_type=jnp.float32)
    ones = _ones_cols(N)
    hlo_ref[...] = jnp.concatenate([h[:, :64], ones], axis=1)
    hhi_ref[...] = jnp.concatenate([h[:, 64:], ones], axis=1)
    alph_ref[...] = jnp.dot(h, a2_ref[...], preferred_element_type=jnp.float32)


def _mid_body(nlo_ref, nhi_ref, w2_ref, a2_ref, h2_ref, alph2_ref):
    slo = nlo_ref[0] + nlo_ref[1]
    shi = nhi_ref[0] + nhi_ref[1]
    den = slo[:, 64:65] + 1e-16
    x1 = jnp.maximum(jnp.concatenate([slo[:, :64], shi[:, :64]], axis=1) / den,
                     0.0)
    h2 = jnp.dot(x1, w2_ref[...], preferred_element_type=jnp.float32)
    h2_ref[...] = jnp.concatenate([h2, _ones_cols(N)], axis=1)
    alph2_ref[...] = jnp.dot(h2, a2_ref[...], preferred_element_type=jnp.float32)


def _fin_body(num_ref, out_ref):
    s = num_ref[0] + num_ref[1]
    out_ref[...] = jnp.maximum(s[:, :D2] / (s[:, D2:D2 + 1] + 1e-16), 0.0)


def _make_scalar_kernel():
    mesh = plsc.VectorSubcoreMesh(core_axis_name="c", subcore_axis_name="s")

    @functools.partial(
        pl.kernel,
        out_type=jax.ShapeDtypeStruct((E_PAD,), jnp.float32),
        mesh=mesh,
        compiler_params=pltpu.CompilerParams(
            use_tc_tiling_on_sc=False, needs_layout_passes=False),
        scratch_types=[
            pltpu.VMEM((PER_TILE,), jnp.int32),
            pltpu.VMEM((PER_TILE,), jnp.int32),
            pltpu.VMEM((PER_TILE,), jnp.float32),
            pltpu.VMEM((N, 2), jnp.float32),
            pltpu.SemaphoreType.DMA,
        ],
    )
    def scalar_kernel(src_hbm, dst_hbm, alph_hbm, ez_hbm,
                      srcv, dstv, ezv, alphv, sem):
        cid = lax.axis_index("c")
        sid = lax.axis_index("s")
        wid = cid * NS + sid
        base_e = wid * PER_TILE

        pltpu.sync_copy(alph_hbm, alphv)
        pltpu.sync_copy(src_hbm.at[pl.ds(base_e, PER_TILE)], srcv)
        pltpu.sync_copy(dst_hbm.at[pl.ds(base_e, PER_TILE)], dstv)

        zero_i = jnp.zeros((L,), jnp.int32)
        one_i = jnp.full((L,), 1, jnp.int32)
        lane = lax.iota(jnp.int32, L)

        @pl.loop(0, GROUPS)
        def _(g):
            off = g * L
            src16 = srcv[pl.ds(off, L)]
            dst16 = dstv[pl.ds(off, L)]
            asv = plsc.load_gather(alphv, [src16, zero_i])
            adv = plsc.load_gather(alphv, [dst16, one_i])
            e = asv + adv
            e = jnp.where(e > 0.0, e, 0.2 * e)
            ez = jnp.exp(e)
            gid = base_e + off + lane
            ezv[pl.ds(off, L)] = jnp.where(gid < E_TOT, ez, 0.0)

        pltpu.sync_copy(ezv, ez_hbm.at[pl.ds(base_e, PER_TILE)])

    return scalar_kernel


def _make_edge_kernel(D):
    DA = D + L
    CCH = DA // L
    mesh = plsc.VectorSubcoreMesh(core_axis_name="c", subcore_axis_name="s")

    @functools.partial(
        pl.kernel,
        out_type=jax.ShapeDtypeStruct((NC, N, DA), jnp.float32),
        mesh=mesh,
        compiler_params=pltpu.CompilerParams(
            use_tc_tiling_on_sc=False, needs_layout_passes=False),
        scratch_types=[
            pltpu.VMEM((CH,), jnp.int32),
            pltpu.VMEM((CH,), jnp.int32),
            pltpu.VMEM((CH,), jnp.float32),
            pltpu.VMEM((L, DA), jnp.float32),
            pltpu.VMEM((ZCH, DA), jnp.float32),
            pltpu.VMEM_SHARED((N, DA), jnp.float32),
            pltpu.SemaphoreType.DMA,
        ],
    )
    def edge_kernel(h_hbm, src_hbm, dst_hbm, ez_hbm, num_hbm,
                    srcv, dstv, ezv, rowb, zb, num_sh, sem):
        cid = lax.axis_index("c")
        sid = lax.axis_index("s")
        wid = cid * NS + sid
        row0 = sid * ROWS_PER_TILE

        zvec = jnp.zeros((L,), jnp.float32)

        @pl.loop(0, ZCH)
        def _(i):
            for c in range(CCH):
                zb[i, pl.ds(c * L, L)] = zvec

        @pl.loop(0, ROWS_PER_TILE // ZCH)
        def _(j):
            pltpu.sync_copy(zb, num_sh.at[pl.ds(row0 + j * ZCH, ZCH)])

        base_e = wid * PER_TILE

        plsc.subcore_barrier()

        @pl.loop(0, NCHUNK)
        def _(cc):
            cbase = base_e + cc * CH
            pltpu.sync_copy(src_hbm.at[pl.ds(cbase, CH)], srcv)
            pltpu.sync_copy(dst_hbm.at[pl.ds(cbase, CH)], dstv)
            pltpu.sync_copy(ez_hbm.at[pl.ds(cbase, CH)], ezv)

            @pl.loop(0, CGROUPS)
            def _(g):
                off = g * L
                src16 = srcv[pl.ds(off, L)]
                dst16 = dstv[pl.ds(off, L)]
                ez = ezv[pl.ds(off, L)]
                pltpu.async_copy(h_hbm.at[src16], rowb, sem).wait()
                for r in range(L):
                    sez = ez[r]
                    for c in range(CCH):
                        sl = pl.ds(c * L, L)
                        rowb[r, sl] = rowb[r, sl] * sez
                pltpu.sync_copy(rowb, num_sh.at[dst16], add=True)

        plsc.subcore_barrier()

        @pl.loop(0, ROWS_PER_TILE // ZCH)
        def _(j):
            r0 = row0 + j * ZCH
            pltpu.sync_copy(num_sh.at[pl.ds(r0, ZCH)],
                            num_hbm.at[cid, pl.ds(r0, ZCH)])

    return edge_kernel


_scalar_k = _make_scalar_kernel()
_edge_k = _make_edge_kernel(D2)

_prep1 = pl.pallas_call(
    _prep1_body,
    out_shape=[
        jax.ShapeDtypeStruct((N, D2 + L), jnp.float32),
        jax.ShapeDtypeStruct((N, D2 + L), jnp.float32),
        jax.ShapeDtypeStruct((N, 2), jnp.float32),
    ],
)

_mid = pl.pallas_call(
    _mid_body,
    out_shape=[
        jax.ShapeDtypeStruct((N, D2 + L), jnp.float32),
        jax.ShapeDtypeStruct((N, 2), jnp.float32),
    ],
)

_fin = pl.pallas_call(
    _fin_body,
    out_shape=jax.ShapeDtypeStruct((N, D2), jnp.float32),
)


def kernel(x, edge_index, W1, a_src1, a_dst1, W2, a_src2, a_dst2):
    ei = edge_index.astype(jnp.int32)
    loop_idx = jnp.arange(N, dtype=jnp.int32)
    pad = jnp.zeros((E_PAD - E_TOT,), jnp.int32)
    src = jnp.concatenate([ei[0], loop_idx, pad])
    dst = jnp.concatenate([ei[1], loop_idx, pad])

    a2_1 = jnp.zeros((D1, 2), jnp.float32).at[:, 0].set(a_src1).at[:, 1].set(a_dst1)
    a2_2 = jnp.zeros((D2, 2), jnp.float32).at[:, 0].set(a_src2).at[:, 1].set(a_dst2)

    h_lo, h_hi, alph1 = _prep1(x, W1, a2_1)
    ez1 = _scalar_k(src, dst, alph1)
    num_lo = _edge_k(h_lo, src, dst, ez1)
    num_hi = _edge_k(h_hi, src, dst, ez1)
    h2, alph2 = _mid(num_lo, num_hi, W2, a2_2)
    ez2 = _scalar_k(src, dst, alph2)
    num2 = _edge_k(h2, src, dst, ez2)
    return _fin(num2)


# speedup vs baseline: 24.4610x; 1.8834x over previous
import functools

import jax
import jax.numpy as jnp
from jax import lax
from jax.experimental import pallas as pl
from jax.experimental.pallas import tpu as pltpu
from jax.experimental.pallas import tpu_sc as plsc

N = 10000
E_RAW = 320000
E_TOT = E_RAW + N
D1 = 128
D2 = 64
DB = 64

NC = 2
NS = 16
L = 16
NW = NC * NS

NBUF = 6
KA = 3
NCHUNK = 2
CGROUPS = 327
CH = CGROUPS * L
PER_TILE = NCHUNK * CH
E_PAD = NW * PER_TILE
GROUPS = PER_TILE // L
ROWS_PER_TILE = N // NS
ZCH = 25


def _prep1_body(x_ref, w_ref, a2_ref, hlo_ref, hhi_ref, alph_ref):
    x = x_ref[...]
    h = jnp.dot(x, w_ref[...], preferred_element_type=jnp.float32)
    hlo_ref[...] = h[:, :DB]
    hhi_ref[...] = h[:, DB:]
    alph_ref[...] = jnp.dot(h, a2_ref[...], preferred_element_type=jnp.float32)


def _mid_body(nlo_ref, nhi_ref, dp_ref, w2_ref, a2_ref, h2_ref, alph2_ref):
    s = jnp.concatenate(
        [nlo_ref[0] + nlo_ref[1], nhi_ref[0] + nhi_ref[1]], axis=1)
    den = dp_ref[0, :, 0:1] + dp_ref[1, :, 0:1] + 1e-16
    x1 = jnp.maximum(s / den, 0.0)
    h2 = jnp.dot(x1, w2_ref[...], preferred_element_type=jnp.float32)
    h2_ref[...] = h2
    alph2_ref[...] = jnp.dot(h2, a2_ref[...], preferred_element_type=jnp.float32)


def _fin_body(num_ref, dp_ref, out_ref):
    s = num_ref[0] + num_ref[1]
    den = dp_ref[0, :, 0:1] + dp_ref[1, :, 0:1] + 1e-16
    out_ref[...] = jnp.maximum(s / den, 0.0)


def _make_scalar_kernel():
    mesh = plsc.VectorSubcoreMesh(core_axis_name="c", subcore_axis_name="s")

    @functools.partial(
        pl.kernel,
        out_type=[
            jax.ShapeDtypeStruct((E_PAD,), jnp.float32),
            jax.ShapeDtypeStruct((NC, N, L), jnp.float32),
        ],
        mesh=mesh,
        compiler_params=pltpu.CompilerParams(
            use_tc_tiling_on_sc=False, needs_layout_passes=False),
        scratch_types=[
            pltpu.VMEM((PER_TILE,), jnp.int32),
            pltpu.VMEM((PER_TILE,), jnp.int32),
            pltpu.VMEM((PER_TILE,), jnp.float32),
            pltpu.VMEM((N, 2), jnp.float32),
            pltpu.VMEM((L, L), jnp.float32),
            pltpu.VMEM((ZCH, L), jnp.float32),
            pltpu.VMEM_SHARED((N, L), jnp.float32),
            pltpu.SemaphoreType.DMA,
        ],
    )
    def scalar_kernel(src_hbm, dst_hbm, alph_hbm, ez_hbm, den_hbm,
                      srcv, dstv, ezv, alphv, denrow, zb, den_sh, sem):
        cid = lax.axis_index("c")
        sid = lax.axis_index("s")
        wid = cid * NS + sid
        base_e = wid * PER_TILE
        row0 = sid * ROWS_PER_TILE

        zvec = jnp.zeros((L,), jnp.float32)

        @pl.loop(0, ZCH)
        def _(i):
            zb[i, pl.ds(0, L)] = zvec

        @pl.loop(0, ROWS_PER_TILE // ZCH)
        def _(j):
            pltpu.sync_copy(zb, den_sh.at[pl.ds(row0 + j * ZCH, ZCH)])

        for r in range(L):
            denrow[r, pl.ds(0, L)] = zvec

        pltpu.sync_copy(alph_hbm, alphv)
        pltpu.sync_copy(src_hbm.at[pl.ds(base_e, PER_TILE)], srcv)
        pltpu.sync_copy(dst_hbm.at[pl.ds(base_e, PER_TILE)], dstv)

        plsc.subcore_barrier()

        zero_i = jnp.zeros((L,), jnp.int32)
        one_i = jnp.full((L,), 1, jnp.int32)
        lane = lax.iota(jnp.int32, L)

        @pl.loop(0, GROUPS)
        def _(g):
            off = g * L
            src16 = srcv[pl.ds(off, L)]
            dst16 = dstv[pl.ds(off, L)]
            asv = plsc.load_gather(alphv, [src16, zero_i])
            adv = plsc.load_gather(alphv, [dst16, one_i])
            e = asv + adv
            e = jnp.where(e > 0.0, e, 0.2 * e)
            ez = jnp.exp(e)
            gid = base_e + off + lane
            ez = jnp.where(gid < E_TOT, ez, 0.0)
            ezv[pl.ds(off, L)] = ez
            plsc.store_scatter(denrow, [lane, zero_i], ez)
            pltpu.sync_copy(denrow, den_sh.at[dst16], add=True)

        pltpu.sync_copy(ezv, ez_hbm.at[pl.ds(base_e, PER_TILE)])

        plsc.subcore_barrier()

        @pl.loop(0, ROWS_PER_TILE // ZCH)
        def _(j):
            r0 = row0 + j * ZCH
            pltpu.sync_copy(den_sh.at[pl.ds(r0, ZCH)],
                            den_hbm.at[cid, pl.ds(r0, ZCH)])

    return scalar_kernel


def _make_edge_kernel():
    mesh = plsc.VectorSubcoreMesh(core_axis_name="c", subcore_axis_name="s")

    @functools.partial(
        pl.kernel,
        out_type=jax.ShapeDtypeStruct((NC, N, DB), jnp.float32),
        mesh=mesh,
        compiler_params=pltpu.CompilerParams(
            use_tc_tiling_on_sc=False, needs_layout_passes=False),
        scratch_types=[
            pltpu.VMEM((CH,), jnp.int32),
            pltpu.VMEM((CH,), jnp.int32),
            pltpu.VMEM((CH,), jnp.float32),
            pltpu.VMEM((NBUF * L, DB), jnp.float32),
            pltpu.VMEM((ZCH, DB), jnp.float32),
            pltpu.VMEM_SHARED((N, DB), jnp.float32),
            pltpu.SemaphoreType.DMA((NBUF,)),
            pltpu.SemaphoreType.DMA((NBUF,)),
        ],
    )
    def edge_kernel(h_hbm, src_hbm, dst_hbm, ez_hbm, num_hbm,
                    srcv, dstv, ezv, rowb, zb, num_sh, gsem, ssem):
        cid = lax.axis_index("c")
        sid = lax.axis_index("s")
        wid = cid * NS + sid
        row0 = sid * ROWS_PER_TILE

        zvec = jnp.zeros((L,), jnp.float32)

        @pl.loop(0, ZCH)
        def _(i):
            for c in range(DB // L):
                zb[i, pl.ds(c * L, L)] = zvec

        @pl.loop(0, ROWS_PER_TILE // ZCH)
        def _(j):
            pltpu.sync_copy(zb, num_sh.at[pl.ds(row0 + j * ZCH, ZCH)])

        base_e = wid * PER_TILE

        plsc.subcore_barrier()

        def issue_gather(off, b):
            src16 = srcv[pl.ds(off, L)]
            pltpu.async_copy(h_hbm.at[src16],
                             rowb.at[pl.ds(b * L, L)], gsem.at[b])

        def wait_gather(b):
            pltpu.make_async_copy(h_hbm.at[pl.ds(0, L)],
                                  rowb.at[pl.ds(b * L, L)],
                                  gsem.at[b]).wait()

        def scale_rows(off, b):
            ez = ezv[pl.ds(off, L)]
            for r in range(L):
                sez = ez[r]
                for c in range(DB // L):
                    sl = pl.ds(c * L, L)
                    rowb[b * L + r, sl] = rowb[b * L + r, sl] * sez

        def issue_scatter(off, b):
            dst16 = dstv[pl.ds(off, L)]
            pltpu.async_copy(rowb.at[pl.ds(b * L, L)],
                             num_sh.at[dst16], ssem.at[b], add=True)

        def wait_scatter(b):
            pltpu.make_async_copy(rowb.at[pl.ds(b * L, L)],
                                  num_sh.at[pl.ds(0, L)],
                                  ssem.at[b]).wait()

        @pl.loop(0, NCHUNK)
        def _(cc):
            cbase = base_e + cc * CH
            pltpu.sync_copy(src_hbm.at[pl.ds(cbase, CH)], srcv)
            pltpu.sync_copy(dst_hbm.at[pl.ds(cbase, CH)], dstv)
            pltpu.sync_copy(ez_hbm.at[pl.ds(cbase, CH)], ezv)

            for b in range(KA):
                issue_gather(b * L, b)

            def do_round(g0, first):
                for k in range(NBUF):
                    g = g0 + k
                    b = k
                    ba = (k + KA) % NBUF
                    if not (first and k < KA):
                        wait_scatter(ba)
                    issue_gather((g + KA) * L, ba)
                    wait_gather(b)
                    scale_rows(g * L, b)
                    issue_scatter(g * L, b)

            do_round(0, True)

            @pl.loop(1, (CGROUPS - KA) // NBUF)
            def _(j):
                do_round(j * NBUF, False)

            for k in range(KA):
                g = CGROUPS - KA + k
                b = g % NBUF
                wait_gather(b)
                scale_rows(g * L, b)
                issue_scatter(g * L, b)

            for b in range(NBUF):
                wait_scatter(b)

        plsc.subcore_barrier()

        @pl.loop(0, ROWS_PER_TILE // ZCH)
        def _(j):
            r0 = row0 + j * ZCH
            pltpu.sync_copy(num_sh.at[pl.ds(r0, ZCH)],
                            num_hbm.at[cid, pl.ds(r0, ZCH)])

    return edge_kernel


_scalar_k = _make_scalar_kernel()
_edge_k = _make_edge_kernel()

_prep1 = pl.pallas_call(
    _prep1_body,
    out_shape=[
        jax.ShapeDtypeStruct((N, DB), jnp.float32),
        jax.ShapeDtypeStruct((N, DB), jnp.float32),
        jax.ShapeDtypeStruct((N, 2), jnp.float32),
    ],
)

_mid = pl.pallas_call(
    _mid_body,
    out_shape=[
        jax.ShapeDtypeStruct((N, D2), jnp.float32),
        jax.ShapeDtypeStruct((N, 2), jnp.float32),
    ],
)

_fin = pl.pallas_call(
    _fin_body,
    out_shape=jax.ShapeDtypeStruct((N, D2), jnp.float32),
)


def kernel(x, edge_index, W1, a_src1, a_dst1, W2, a_src2, a_dst2):
    ei = edge_index.astype(jnp.int32)
    loop_idx = jnp.arange(N, dtype=jnp.int32)
    pad = jnp.zeros((E_PAD - E_TOT,), jnp.int32)
    src = jnp.concatenate([ei[0], loop_idx, pad])
    dst = jnp.concatenate([ei[1], loop_idx, pad])

    a2_1 = jnp.zeros((D1, 2), jnp.float32).at[:, 0].set(a_src1).at[:, 1].set(a_dst1)
    a2_2 = jnp.zeros((D2, 2), jnp.float32).at[:, 0].set(a_src2).at[:, 1].set(a_dst2)

    h_lo, h_hi, alph1 = _prep1(x, W1, a2_1)
    ez1, dp1 = _scalar_k(src, dst, alph1)
    num_lo = _edge_k(h_lo, src, dst, ez1)
    num_hi = _edge_k(h_hi, src, dst, ez1)
    h2, alph2 = _mid(num_lo, num_hi, dp1, W2, a2_2)
    ez2, dp2 = _scalar_k(src, dst, alph2)
    num2 = _edge_k(h2, src, dst, ez2)
    return _fin(num2, dp2)


# speedup vs baseline: 35.7136x; 1.4600x over previous
import functools

import jax
import jax.numpy as jnp
from jax import lax
from jax.experimental import pallas as pl
from jax.experimental.pallas import tpu as pltpu
from jax.experimental.pallas import tpu_sc as plsc

N = 10000
E_RAW = 320000
E_TOT = E_RAW + N
D1 = 128
D2 = 64
DB = 64

NC = 2
NS = 16
L = 16
NW = NC * NS

NBUF = 8
KA = 4
NCHUNK = 2
CGROUPS = 324
CH = CGROUPS * L
PER_TILE = NCHUNK * CH
E_PAD = NW * PER_TILE
GROUPS = PER_TILE // L
NROW128 = PER_TILE // 128
ROWS_PER_TILE = N // NS
ZCH = 25
ND = 10240


def _prep1_body(x_ref, w_ref, a2_ref, hlo_ref, hhi_ref, alph_ref):
    x = x_ref[...]
    h = jnp.dot(x, w_ref[...], preferred_element_type=jnp.float32)
    hlo_ref[...] = h[:, :DB]
    hhi_ref[...] = h[:, DB:]
    alph_ref[...] = jnp.dot(h, a2_ref[...], preferred_element_type=jnp.float32)


def _mid_body(nlo_ref, nhi_ref, dp_ref, w2_ref, a2_ref, h2_ref, alph2_ref):
    s = jnp.concatenate(
        [nlo_ref[0] + nlo_ref[1], nhi_ref[0] + nhi_ref[1]], axis=1)
    den = jnp.reshape(dp_ref[0, :N] + dp_ref[1, :N], (N, 1)) + 1e-16
    x1 = jnp.maximum(s / den, 0.0)
    h2 = jnp.dot(x1, w2_ref[...], preferred_element_type=jnp.float32)
    h2_ref[...] = h2
    alph2_ref[...] = jnp.dot(h2, a2_ref[...], preferred_element_type=jnp.float32)


def _fin_body(num_ref, dp_ref, out_ref):
    s = num_ref[0] + num_ref[1]
    den = jnp.reshape(dp_ref[0, :N] + dp_ref[1, :N], (N, 1)) + 1e-16
    out_ref[...] = jnp.maximum(s / den, 0.0)


def _make_scalar_kernel():
    mesh = plsc.VectorSubcoreMesh(core_axis_name="c", subcore_axis_name="s")

    @functools.partial(
        pl.kernel,
        out_type=[
            jax.ShapeDtypeStruct((NW * NROW128, 128), jnp.float32),
            jax.ShapeDtypeStruct((NC, ND), jnp.float32),
        ],
        mesh=mesh,
        compiler_params=pltpu.CompilerParams(
            use_tc_tiling_on_sc=False, needs_layout_passes=False),
        scratch_types=[
            pltpu.VMEM((PER_TILE,), jnp.int32),
            pltpu.VMEM((NROW128, 128), jnp.int32),
            pltpu.VMEM((NROW128, 128), jnp.float32),
            pltpu.VMEM((N, 2), jnp.float32),
            pltpu.VMEM((ND // NS,), jnp.float32),
            pltpu.VMEM_SHARED((ND,), jnp.float32),
            pltpu.SemaphoreType.DMA,
            pltpu.SemaphoreType.DMA,
        ],
    )
    def scalar_kernel(src_hbm, dst2_hbm, alph_hbm, ez_hbm, den_hbm,
                      srcv, dstm, ezm, alphv, zden, den_sh, sem, dsem):
        cid = lax.axis_index("c")
        sid = lax.axis_index("s")
        wid = cid * NS + sid
        base_e = wid * PER_TILE
        drow0 = sid * (ND // NS)

        zvec = jnp.zeros((L,), jnp.float32)

        @pl.loop(0, ND // NS // L)
        def _(i):
            zden[pl.ds(i * L, L)] = zvec

        pltpu.sync_copy(zden, den_sh.at[pl.ds(drow0, ND // NS)])

        pltpu.sync_copy(alph_hbm, alphv)
        pltpu.sync_copy(src_hbm.at[pl.ds(base_e, PER_TILE)], srcv)
        pltpu.sync_copy(dst2_hbm.at[pl.ds(wid * NROW128, NROW128)], dstm)

        plsc.subcore_barrier()

        zero_i = jnp.zeros((L,), jnp.int32)
        one_i = jnp.full((L,), 1, jnp.int32)
        lane = lax.iota(jnp.int32, L)

        @pl.loop(0, NROW128)
        def _(jr):
            for gi in range(8):
                off = (jr * 8 + gi) * L
                src16 = srcv[pl.ds(off, L)]
                dst16 = dstm[jr, pl.ds(gi * L, L)]
                asv = plsc.load_gather(alphv, [src16, zero_i])
                adv = plsc.load_gather(alphv, [dst16, one_i])
                e = asv + adv
                e = jnp.where(e > 0.0, e, 0.2 * e)
                ez = jnp.exp(e)
                gid = base_e + off + lane
                ezm[jr, pl.ds(gi * L, L)] = jnp.where(gid < E_TOT, ez, 0.0)
            pltpu.async_copy(ezm.at[jr], den_sh.at[dstm.at[jr]], dsem,
                             add=True)

        pltpu.sync_copy(ezm, ez_hbm.at[pl.ds(wid * NROW128, NROW128)])

        @pl.loop(0, NROW128)
        def _(jr):
            pltpu.make_async_copy(ezm.at[0], den_sh.at[pl.ds(0, 128)],
                                  dsem).wait()

        plsc.subcore_barrier()

        pltpu.sync_copy(den_sh.at[pl.ds(drow0, ND // NS)],
                        den_hbm.at[cid, pl.ds(drow0, ND // NS)])

    return scalar_kernel


def _make_edge_kernel():
    mesh = plsc.VectorSubcoreMesh(core_axis_name="c", subcore_axis_name="s")

    @functools.partial(
        pl.kernel,
        out_type=jax.ShapeDtypeStruct((NC, N, DB), jnp.float32),
        mesh=mesh,
        compiler_params=pltpu.CompilerParams(
            use_tc_tiling_on_sc=False, needs_layout_passes=False),
        scratch_types=[
            pltpu.VMEM((CH,), jnp.int32),
            pltpu.VMEM((CH,), jnp.int32),
            pltpu.VMEM((CH,), jnp.float32),
            pltpu.VMEM((NBUF * L, DB), jnp.float32),
            pltpu.VMEM((ZCH, DB), jnp.float32),
            pltpu.VMEM_SHARED((N, DB), jnp.float32),
            pltpu.SemaphoreType.DMA((NBUF,)),
            pltpu.SemaphoreType.DMA((NBUF,)),
        ],
    )
    def edge_kernel(h_hbm, src_hbm, dst_hbm, ez_hbm, num_hbm,
                    srcv, dstv, ezv, rowb, zb, num_sh, gsem, ssem):
        cid = lax.axis_index("c")
        sid = lax.axis_index("s")
        wid = cid * NS + sid
        row0 = sid * ROWS_PER_TILE

        zvec = jnp.zeros((L,), jnp.float32)

        @pl.loop(0, ZCH)
        def _(i):
            for c in range(DB // L):
                zb[i, pl.ds(c * L, L)] = zvec

        @pl.loop(0, ROWS_PER_TILE // ZCH)
        def _(j):
            pltpu.sync_copy(zb, num_sh.at[pl.ds(row0 + j * ZCH, ZCH)])

        base_e = wid * PER_TILE

        plsc.subcore_barrier()

        def issue_gather(off, b):
            src16 = srcv[pl.ds(off, L)]
            pltpu.async_copy(h_hbm.at[src16],
                             rowb.at[pl.ds(b * L, L)], gsem.at[b])

        def wait_gather(b):
            pltpu.make_async_copy(h_hbm.at[pl.ds(0, L)],
                                  rowb.at[pl.ds(b * L, L)],
                                  gsem.at[b]).wait()

        def scale_rows(off, b):
            ez = ezv[pl.ds(off, L)]
            for r in range(L):
                sez = ez[r]
                for c in range(DB // L):
                    sl = pl.ds(c * L, L)
                    rowb[b * L + r, sl] = rowb[b * L + r, sl] * sez

        def issue_scatter(off, b):
            dst16 = dstv[pl.ds(off, L)]
            pltpu.async_copy(rowb.at[pl.ds(b * L, L)],
                             num_sh.at[dst16], ssem.at[b], add=True)

        def wait_scatter(b):
            pltpu.make_async_copy(rowb.at[pl.ds(b * L, L)],
                                  num_sh.at[pl.ds(0, L)],
                                  ssem.at[b]).wait()

        @pl.loop(0, NCHUNK)
        def _(cc):
            cbase = base_e + cc * CH
            pltpu.sync_copy(src_hbm.at[pl.ds(cbase, CH)], srcv)
            pltpu.sync_copy(dst_hbm.at[pl.ds(cbase, CH)], dstv)
            pltpu.sync_copy(ez_hbm.at[pl.ds(cbase, CH)], ezv)

            for b in range(KA):
                issue_gather(b * L, b)

            def do_round(g0, first):
                for k in range(NBUF):
                    g = g0 + k
                    b = k
                    ba = (k + KA) % NBUF
                    if not (first and k < KA):
                        wait_scatter(ba)
                    issue_gather((g + KA) * L, ba)
                    wait_gather(b)
                    scale_rows(g * L, b)
                    issue_scatter(g * L, b)

            do_round(0, True)

            @pl.loop(1, (CGROUPS - KA) // NBUF)
            def _(j):
                do_round(j * NBUF, False)

            for k in range(KA):
                g = CGROUPS - KA + k
                b = g % NBUF
                wait_gather(b)
                scale_rows(g * L, b)
                issue_scatter(g * L, b)

            for b in range(NBUF):
                wait_scatter(b)

        plsc.subcore_barrier()

        @pl.loop(0, ROWS_PER_TILE // ZCH)
        def _(j):
            r0 = row0 + j * ZCH
            pltpu.sync_copy(num_sh.at[pl.ds(r0, ZCH)],
                            num_hbm.at[cid, pl.ds(r0, ZCH)])

    return edge_kernel


_scalar_k = _make_scalar_kernel()
_edge_k = _make_edge_kernel()

_prep1 = pl.pallas_call(
    _prep1_body,
    out_shape=[
        jax.ShapeDtypeStruct((N, DB), jnp.float32),
        jax.ShapeDtypeStruct((N, DB), jnp.float32),
        jax.ShapeDtypeStruct((N, 2), jnp.float32),
    ],
)

_mid = pl.pallas_call(
    _mid_body,
    out_shape=[
        jax.ShapeDtypeStruct((N, D2), jnp.float32),
        jax.ShapeDtypeStruct((N, 2), jnp.float32),
    ],
)

_fin = pl.pallas_call(
    _fin_body,
    out_shape=jax.ShapeDtypeStruct((N, D2), jnp.float32),
)


def kernel(x, edge_index, W1, a_src1, a_dst1, W2, a_src2, a_dst2):
    ei = edge_index.astype(jnp.int32)
    loop_idx = jnp.arange(N, dtype=jnp.int32)
    pad = jnp.zeros((E_PAD - E_TOT,), jnp.int32)
    src = jnp.concatenate([ei[0], loop_idx, pad])
    dst = jnp.concatenate([ei[1], loop_idx, pad])

    a2_1 = jnp.zeros((D1, 2), jnp.float32).at[:, 0].set(a_src1).at[:, 1].set(a_dst1)
    a2_2 = jnp.zeros((D2, 2), jnp.float32).at[:, 0].set(a_src2).at[:, 1].set(a_dst2)

    dst2 = dst.reshape(NW * NROW128, 128)

    h_lo, h_hi, alph1 = _prep1(x, W1, a2_1)
    ez1, dp1 = _scalar_k(src, dst2, alph1)
    ez1 = ez1.reshape(E_PAD)
    num_lo = _edge_k(h_lo, src, dst, ez1)
    num_hi = _edge_k(h_hi, src, dst, ez1)
    h2, alph2 = _mid(num_lo, num_hi, dp1, W2, a2_2)
    ez2, dp2 = _scalar_k(src, dst2, alph2)
    ez2 = ez2.reshape(E_PAD)
    num2 = _edge_k(h2, src, dst, ez2)
    return _fin(num2, dp2)
